# baseline (speedup 1.0000x reference)
import jax
import jax.numpy as jnp
from jax.experimental import pallas as pl


def _chamfer_body(xa_ref, ya_ref, dist_ref, idx_ref):
    xa = xa_ref[0]
    ya = ya_ref[0]
    nt = xa.shape[0]
    m = ya.shape[1]
    t = jnp.dot(xa, ya, preferred_element_type=jnp.float32,
                precision=jax.lax.Precision.HIGHEST)
    mn = jnp.min(t, axis=1, keepdims=True)
    iota = jax.lax.broadcasted_iota(jnp.int32, (nt, m), 1)
    idx = jnp.min(jnp.where(t == mn, iota, jnp.int32(m)), axis=1)
    c0 = xa[:, 0:1]
    c1 = xa[:, 1:2]
    c2 = xa[:, 2:3]
    xx = 0.25 * (c0 * c0 + c1 * c1 + c2 * c2)
    dist_ref[0, 0, 0] = (mn + xx)[:, 0]
    idx_ref[0, 0, 0] = idx


def kernel(input1, input2):
    b, n, _ = input1.shape
    m = input2.shape[1]
    nt = min(256, n)
    n_tiles = n // nt
    ones = jnp.ones((b, n, 1), dtype=jnp.float32)
    xa = jnp.concatenate([-2.0 * input1, ones], axis=2)
    yt = jnp.transpose(input2, (0, 2, 1))
    yy = jnp.sum(input2 * input2, axis=2, keepdims=True)
    ya = jnp.concatenate([yt, jnp.transpose(yy, (0, 2, 1))], axis=1)
    dist, idx = pl.pallas_call(
        _chamfer_body,
        grid=(b, n_tiles),
        in_specs=[
            pl.BlockSpec((1, nt, 4), lambda bi, i: (bi, i, 0)),
            pl.BlockSpec((1, 4, m), lambda bi, i: (bi, 0, 0)),
        ],
        out_specs=[
            pl.BlockSpec((1, 1, 1, nt), lambda bi, i: (bi, i, 0, 0)),
            pl.BlockSpec((1, 1, 1, nt), lambda bi, i: (bi, i, 0, 0)),
        ],
        out_shape=[
            jax.ShapeDtypeStruct((b, n_tiles, 1, nt), jnp.float32),
            jax.ShapeDtypeStruct((b, n_tiles, 1, nt), jnp.int32),
        ],
    )(xa, ya)
    return dist.reshape(b, n), idx.reshape(b, n)


# speedup vs baseline: 1.5878x; 1.5878x over previous
import jax
import jax.numpy as jnp
from jax.experimental import pallas as pl


def _chamfer_body(x_ref, yt_ref, dist_ref, idx_ref):
    x = x_ref[0]
    yt = yt_ref[0]
    nt = x.shape[0]
    m = yt.shape[1]
    d0 = x[:, 0:1] - yt[0:1, :]
    d1 = x[:, 1:2] - yt[1:2, :]
    d2 = x[:, 2:3] - yt[2:3, :]
    d = d0 * d0 + d1 * d1 + d2 * d2
    mn = jnp.min(d, axis=1, keepdims=True)
    iota = jax.lax.broadcasted_iota(jnp.int32, (nt, m), 1).astype(jnp.float32)
    idx_f = jnp.min(jnp.where(d == mn, iota, jnp.float32(m)), axis=1)
    dist_ref[0, 0, 0] = mn[:, 0]
    idx_ref[0, 0, 0] = idx_f.astype(jnp.int32)


def kernel(input1, input2):
    b, n, _ = input1.shape
    m = input2.shape[1]
    nt = min(256, n)
    n_tiles = n // nt
    yt = jnp.transpose(input2, (0, 2, 1))
    dist, idx = pl.pallas_call(
        _chamfer_body,
        grid=(b, n_tiles),
        in_specs=[
            pl.BlockSpec((1, nt, 3), lambda bi, i: (bi, i, 0)),
            pl.BlockSpec((1, 3, m), lambda bi, i: (bi, 0, 0)),
        ],
        out_specs=[
            pl.BlockSpec((1, 1, 1, nt), lambda bi, i: (bi, i, 0, 0)),
            pl.BlockSpec((1, 1, 1, nt), lambda bi, i: (bi, i, 0, 0)),
        ],
        out_shape=[
            jax.ShapeDtypeStruct((b, n_tiles, 1, nt), jnp.float32),
            jax.ShapeDtypeStruct((b, n_tiles, 1, nt), jnp.int32),
        ],
    )(input1, yt)
    return dist.reshape(b, n), idx.reshape(b, n)


# speedup vs baseline: 2.2861x; 1.4397x over previous
import jax
import jax.numpy as jnp
from jax.experimental import pallas as pl


def _chamfer_body(x_ref, ya_ref, dist_ref, idx_ref):
    x = x_ref[0]
    ya = ya_ref[0]
    nt = x.shape[0]
    m = ya.shape[1]
    t = (x[:, 0:1] * ya[0:1, :] + ya[3:4, :]
         + x[:, 1:2] * ya[1:2, :]
         + x[:, 2:3] * ya[2:3, :])
    mn = jnp.min(t, axis=1, keepdims=True)
    iota = jax.lax.broadcasted_iota(jnp.int32, (nt, m), 1).astype(jnp.float32)
    idx_f = jnp.min(jnp.where(t == mn, iota, jnp.float32(m)), axis=1)
    c0 = x[:, 0:1]
    c1 = x[:, 1:2]
    c2 = x[:, 2:3]
    xx = c0 * c0 + c1 * c1 + c2 * c2
    dist_ref[0, 0, 0] = (mn + xx)[:, 0]
    idx_ref[0, 0, 0] = idx_f.astype(jnp.int32)


def kernel(input1, input2):
    b, n, _ = input1.shape
    m = input2.shape[1]
    nt = min(256, n)
    n_tiles = n // nt
    yt = jnp.transpose(input2, (0, 2, 1))
    yy = jnp.sum(input2 * input2, axis=2)[:, None, :]
    ya = jnp.concatenate([-2.0 * yt, yy], axis=1)
    dist, idx = pl.pallas_call(
        _chamfer_body,
        grid=(b, n_tiles),
        in_specs=[
            pl.BlockSpec((1, nt, 3), lambda bi, i: (bi, i, 0)),
            pl.BlockSpec((1, 4, m), lambda bi, i: (bi, 0, 0)),
        ],
        out_specs=[
            pl.BlockSpec((1, 1, 1, nt), lambda bi, i: (bi, i, 0, 0)),
            pl.BlockSpec((1, 1, 1, nt), lambda bi, i: (bi, i, 0, 0)),
        ],
        out_shape=[
            jax.ShapeDtypeStruct((b, n_tiles, 1, nt), jnp.float32),
            jax.ShapeDtypeStruct((b, n_tiles, 1, nt), jnp.int32),
        ],
    )(input1, ya)
    return dist.reshape(b, n), idx.reshape(b, n)


# speedup vs baseline: 2.3049x; 1.0083x over previous
import jax
import jax.numpy as jnp
from jax.experimental import pallas as pl


def _chamfer_body(x_ref, ya_ref, dist_ref, idx_ref):
    x = x_ref[0]
    ya = ya_ref[0]
    nt = x.shape[0]
    m = ya.shape[1]
    t = (x[:, 0:1] * ya[0:1, :] + ya[3:4, :]
         + x[:, 1:2] * ya[1:2, :]
         + x[:, 2:3] * ya[2:3, :])
    mn = jnp.min(t, axis=1, keepdims=True)
    iota = jax.lax.broadcasted_iota(jnp.int32, (nt, m), 1).astype(jnp.float32)
    idx_f = jnp.min(jnp.where(t == mn, iota, jnp.float32(m)), axis=1)
    c0 = x[:, 0:1]
    c1 = x[:, 1:2]
    c2 = x[:, 2:3]
    xx = c0 * c0 + c1 * c1 + c2 * c2
    dist_ref[0, 0, 0] = (mn + xx)[:, 0]
    idx_ref[0, 0, 0] = idx_f.astype(jnp.int32)


def kernel(input1, input2):
    b, n, _ = input1.shape
    m = input2.shape[1]
    nt = min(512, n)
    n_tiles = n // nt
    yt = jnp.transpose(input2, (0, 2, 1))
    yy = jnp.sum(input2 * input2, axis=2)[:, None, :]
    ya = jnp.concatenate([-2.0 * yt, yy], axis=1)
    dist, idx = pl.pallas_call(
        _chamfer_body,
        grid=(b, n_tiles),
        in_specs=[
            pl.BlockSpec((1, nt, 3), lambda bi, i: (bi, i, 0)),
            pl.BlockSpec((1, 4, m), lambda bi, i: (bi, 0, 0)),
        ],
        out_specs=[
            pl.BlockSpec((1, 1, 1, nt), lambda bi, i: (bi, i, 0, 0)),
            pl.BlockSpec((1, 1, 1, nt), lambda bi, i: (bi, i, 0, 0)),
        ],
        out_shape=[
            jax.ShapeDtypeStruct((b, n_tiles, 1, nt), jnp.float32),
            jax.ShapeDtypeStruct((b, n_tiles, 1, nt), jnp.int32),
        ],
    )(input1, ya)
    return dist.reshape(b, n), idx.reshape(b, n)
